# baseline (speedup 1.0000x reference)
import functools

import jax
import jax.numpy as jnp
from jax import lax
from jax.experimental import pallas as pl
from jax.experimental.pallas import tpu as pltpu
from jax.experimental.pallas import tpu_sc as plsc

N = 10000
NP = 10240
E = 160000
EP = 172032
NTILES = 32
NCHUNK = EP // NTILES // 128
ROWS_PT = NP // 16
D_FEAT = 128
D_HID = 2048
D_OUT = 64
BLK = 512
GRID = NP // BLK
MBLK = 1024

_SC_PARAMS = pltpu.CompilerParams(use_tc_tiling_on_sc=False)


@functools.cache
def _make_deg():
    @functools.partial(
        pl.kernel,
        out_type=jax.ShapeDtypeStruct((2, NP, 16), jnp.float32),
        mesh=plsc.VectorSubcoreMesh(core_axis_name="c", subcore_axis_name="s"),
        compiler_params=_SC_PARAMS,
        scratch_types=[
            pltpu.VMEM((NCHUNK, 128), jnp.int32),
            pltpu.VMEM((128, 16), jnp.float32),
            pltpu.VMEM_SHARED((NP, 16), jnp.float32),
            pltpu.SemaphoreType.DMA,
        ],
    )
    def deg_kernel(dst_hbm, ones_hbm, zeros_hbm, out_hbm, dst_v, ones_v, acc,
                   sem_s):
        cid = lax.axis_index("c")
        sid = lax.axis_index("s")
        wid = cid * 16 + sid
        pltpu.sync_copy(zeros_hbm.at[pl.ds(sid * ROWS_PT, ROWS_PT)],
                        acc.at[pl.ds(sid * ROWS_PT, ROWS_PT)])
        pltpu.sync_copy(dst_hbm.at[wid], dst_v)
        pltpu.sync_copy(ones_hbm, ones_v)
        plsc.subcore_barrier()
        for b in range(6):
            pltpu.async_copy(ones_v, acc.at[dst_v.at[b]], sem_s, add=True)

        def body(g, carry):
            c = 6 * g
            for b in range(6):
                pltpu.make_async_copy(ones_v, acc.at[dst_v.at[0]], sem_s).wait()
                pltpu.async_copy(ones_v, acc.at[dst_v.at[c + 6 + b]], sem_s,
                                 add=True)
            return carry

        lax.fori_loop(0, NCHUNK // 6 - 1, body, 0)
        for b in range(6):
            pltpu.make_async_copy(ones_v, acc.at[dst_v.at[0]], sem_s).wait()
        plsc.subcore_barrier()
        pltpu.sync_copy(acc.at[pl.ds(sid * ROWS_PT, ROWS_PT)],
                        out_hbm.at[cid, pl.ds(sid * ROWS_PT, ROWS_PT)])

    return deg_kernel


@functools.cache
def _make_agg(depth, ch, ss):
    nch = EP // NTILES // ch

    @functools.partial(
        pl.kernel,
        out_type=jax.ShapeDtypeStruct((2, NP, depth), jnp.float32),
        mesh=plsc.VectorSubcoreMesh(core_axis_name="c", subcore_axis_name="s"),
        compiler_params=_SC_PARAMS,
        scratch_types=[
            pltpu.VMEM((nch, ch), jnp.int32),
            pltpu.VMEM((nch, ch), jnp.int32),
            [pltpu.VMEM((ch, depth), jnp.float32) for _ in range(2 * ss)],
            pltpu.VMEM_SHARED((NP, depth), jnp.float32),
            pltpu.SemaphoreType.DMA,
            pltpu.SemaphoreType.DMA,
            pltpu.SemaphoreType.DMA,
            pltpu.SemaphoreType.DMA,
        ],
    )
    def agg(src_hbm, dst_hbm, feat_hbm, zeros_hbm, out_hbm,
            src_v, dst_v, bufs, acc, sem_ga, sem_gb, sem_sa, sem_sb):
        cid = lax.axis_index("c")
        sid = lax.axis_index("s")
        wid = cid * 16 + sid
        buf_a, buf_b = bufs[0:ss], bufs[ss:2 * ss]
        pltpu.sync_copy(zeros_hbm.at[pl.ds(sid * ROWS_PT, ROWS_PT)],
                        acc.at[pl.ds(sid * ROWS_PT, ROWS_PT)])
        pltpu.sync_copy(src_hbm.at[wid], src_v)
        pltpu.sync_copy(dst_hbm.at[wid], dst_v)
        for b in range(ss):
            pltpu.async_copy(feat_hbm.at[src_v.at[b]], buf_a[b], sem_ga)
        for b in range(ss):
            pltpu.async_copy(feat_hbm.at[src_v.at[ss + b]], buf_b[b], sem_gb)
        plsc.subcore_barrier()

        def body(g, carry):
            c = 2 * ss * g
            for b in range(ss):
                pltpu.make_async_copy(feat_hbm.at[src_v.at[0]], buf_a[b],
                                      sem_ga).wait()
                pltpu.async_copy(buf_a[b], acc.at[dst_v.at[c + b]], sem_sa,
                                 add=True)
            for b in range(ss):
                pltpu.make_async_copy(feat_hbm.at[src_v.at[0]], buf_b[b],
                                      sem_gb).wait()
                pltpu.async_copy(buf_b[b], acc.at[dst_v.at[c + ss + b]],
                                 sem_sb, add=True)
            for b in range(ss):
                pltpu.make_async_copy(buf_a[b], acc.at[dst_v.at[0]],
                                      sem_sa).wait()
            for b in range(ss):
                nxt = jnp.minimum(c + 2 * ss + b, nch - 1)
                pltpu.async_copy(feat_hbm.at[src_v.at[nxt]], buf_a[b], sem_ga)
            for b in range(ss):
                pltpu.make_async_copy(buf_b[b], acc.at[dst_v.at[0]],
                                      sem_sb).wait()
            for b in range(ss):
                nxt = jnp.minimum(c + 3 * ss + b, nch - 1)
                pltpu.async_copy(feat_hbm.at[src_v.at[nxt]], buf_b[b], sem_gb)
            return carry

        lax.fori_loop(0, nch // (2 * ss), body, 0)
        for b in range(ss):
            pltpu.make_async_copy(feat_hbm.at[src_v.at[0]], buf_a[b],
                                  sem_ga).wait()
            pltpu.make_async_copy(feat_hbm.at[src_v.at[0]], buf_b[b],
                                  sem_gb).wait()
        plsc.subcore_barrier()
        pltpu.sync_copy(acc.at[pl.ds(sid * ROWS_PT, ROWS_PT)],
                        out_hbm.at[cid, pl.ds(sid * ROWS_PT, ROWS_PT)])

    return agg


def _scale_body(d_ref, x_ref, s_ref, xs_ref):
    deg = d_ref[0][:, 0:1] + d_ref[1][:, 0:1] + 1.0
    s = lax.rsqrt(deg)
    s_ref[...] = s
    xs_ref[...] = x_ref[...] * s


def _mlp_body(a_ref, xs_ref, s_ref, w1_ref, b1_ref, w2_ref, ps_ref):
    s = s_ref[...]
    hin = (a_ref[0] + a_ref[1] + xs_ref[...]) * s
    h = jnp.dot(hin.astype(jnp.bfloat16), w1_ref[...],
                preferred_element_type=jnp.float32)
    h = jnp.maximum(h + b1_ref[...], 0.0)
    p = jnp.dot(h.astype(jnp.bfloat16), w2_ref[...],
                preferred_element_type=jnp.float32)
    ps_ref[...] = p * s


def _out_body(a_ref, ps_ref, s_ref, b2_ref, o_ref):
    o_ref[...] = ((a_ref[0, :N] + a_ref[1, :N] + ps_ref[:N]) * s_ref[:N]
                  + b2_ref[...])


def kernel(x, edge_index, W1, b1, W2, b2):
    src = edge_index[0]
    dst = edge_index[1]
    padi = (jnp.arange(EP - E, dtype=jnp.int32) % (NP - N)) + N
    srcf = jnp.concatenate([src, padi])
    dstf = jnp.concatenate([dst, padi])
    srcp1 = srcf.reshape(NTILES, EP // NTILES // 64, 64)
    dstp1 = dstf.reshape(NTILES, EP // NTILES // 64, 64)
    srcp2 = srcf.reshape(NTILES, NCHUNK, 128)
    dstp2 = dstf.reshape(NTILES, NCHUNK, 128)
    xp = jnp.zeros((NP, D_FEAT), jnp.float32).at[:N].set(x)
    zf = jnp.zeros((NP, D_FEAT), jnp.float32)
    z16 = jnp.zeros((NP, 16), jnp.float32)
    zo = jnp.zeros((NP, D_OUT), jnp.float32)

    deg = _make_deg()(dstp2, jnp.ones((128, 16), jnp.float32), z16)

    s_col, xs = pl.pallas_call(
        _scale_body,
        out_shape=[
            jax.ShapeDtypeStruct((NP, 1), jnp.float32),
            jax.ShapeDtypeStruct((NP, D_FEAT), jnp.float32),
        ],
    )(deg, xp)

    agg1 = _make_agg(D_FEAT, 64, 2)(srcp1, dstp1, xs, zf)

    ps = pl.pallas_call(
        _mlp_body,
        grid=(NP // MBLK,),
        in_specs=[
            pl.BlockSpec((2, MBLK, D_FEAT), lambda i: (0, i, 0)),
            pl.BlockSpec((MBLK, D_FEAT), lambda i: (i, 0)),
            pl.BlockSpec((MBLK, 1), lambda i: (i, 0)),
            pl.BlockSpec((D_FEAT, D_HID), lambda i: (0, 0)),
            pl.BlockSpec((1, D_HID), lambda i: (0, 0)),
            pl.BlockSpec((D_HID, D_OUT), lambda i: (0, 0)),
        ],
        out_specs=pl.BlockSpec((MBLK, D_OUT), lambda i: (i, 0)),
        out_shape=jax.ShapeDtypeStruct((NP, D_OUT), jnp.float32),
    )(agg1, xs, s_col, W1.astype(jnp.bfloat16), b1.reshape(1, D_HID),
      W2.astype(jnp.bfloat16))

    agg2 = _make_agg(D_OUT, 128, 3)(srcp2, dstp2, ps, zo)

    out = pl.pallas_call(
        _out_body,
        out_shape=jax.ShapeDtypeStruct((N, D_OUT), jnp.float32),
    )(agg2, ps, s_col, b2.reshape(1, D_OUT))

    return out


# speedup vs baseline: 1.0158x; 1.0158x over previous
import functools

import jax
import jax.numpy as jnp
from jax import lax
from jax.experimental import pallas as pl
from jax.experimental.pallas import tpu as pltpu
from jax.experimental.pallas import tpu_sc as plsc

N = 10000
NP = 10240
E = 160000
EP = 172032
NTILES = 32
NCHUNK = EP // NTILES // 128
ROWS_PT = NP // 16
D_FEAT = 128
D_HID = 2048
D_OUT = 64
BLK = 512
GRID = NP // BLK
MBLK = 1024

_SC_PARAMS = pltpu.CompilerParams(use_tc_tiling_on_sc=False)


@functools.cache
def _make_deg():
    @functools.partial(
        pl.kernel,
        out_type=jax.ShapeDtypeStruct((2, NP, 16), jnp.float32),
        mesh=plsc.VectorSubcoreMesh(core_axis_name="c", subcore_axis_name="s"),
        compiler_params=_SC_PARAMS,
        scratch_types=[
            pltpu.VMEM((NCHUNK, 128), jnp.int32),
            pltpu.VMEM((128, 16), jnp.float32),
            pltpu.VMEM_SHARED((NP, 16), jnp.float32),
            pltpu.SemaphoreType.DMA,
        ],
    )
    def deg_kernel(dst_hbm, ones_hbm, zeros_hbm, out_hbm, dst_v, ones_v, acc,
                   sem_s):
        cid = lax.axis_index("c")
        sid = lax.axis_index("s")
        wid = cid * 16 + sid
        pltpu.sync_copy(zeros_hbm.at[pl.ds(sid * ROWS_PT, ROWS_PT)],
                        acc.at[pl.ds(sid * ROWS_PT, ROWS_PT)])
        pltpu.sync_copy(dst_hbm.at[wid], dst_v)
        pltpu.sync_copy(ones_hbm, ones_v)
        plsc.subcore_barrier()
        for b in range(6):
            pltpu.async_copy(ones_v, acc.at[dst_v.at[b]], sem_s, add=True)

        def body(g, carry):
            c = 6 * g
            for b in range(6):
                pltpu.make_async_copy(ones_v, acc.at[dst_v.at[0]], sem_s).wait()
                pltpu.async_copy(ones_v, acc.at[dst_v.at[c + 6 + b]], sem_s,
                                 add=True)
            return carry

        lax.fori_loop(0, NCHUNK // 6 - 1, body, 0)
        for b in range(6):
            pltpu.make_async_copy(ones_v, acc.at[dst_v.at[0]], sem_s).wait()
        plsc.subcore_barrier()
        pltpu.sync_copy(acc.at[pl.ds(sid * ROWS_PT, ROWS_PT)],
                        out_hbm.at[cid, pl.ds(sid * ROWS_PT, ROWS_PT)])

    return deg_kernel


@functools.cache
def _make_agg(depth, ch, ss):
    nch = EP // NTILES // ch

    @functools.partial(
        pl.kernel,
        out_type=jax.ShapeDtypeStruct((2, NP, depth), jnp.float32),
        mesh=plsc.VectorSubcoreMesh(core_axis_name="c", subcore_axis_name="s"),
        compiler_params=_SC_PARAMS,
        scratch_types=[
            pltpu.VMEM((nch, ch), jnp.int32),
            pltpu.VMEM((nch, ch), jnp.int32),
            [pltpu.VMEM((ch, depth), jnp.float32) for _ in range(2 * ss)],
            pltpu.VMEM_SHARED((NP, depth), jnp.float32),
            pltpu.SemaphoreType.DMA,
            pltpu.SemaphoreType.DMA,
            pltpu.SemaphoreType.DMA,
            pltpu.SemaphoreType.DMA,
        ],
    )
    def agg(src_hbm, dst_hbm, feat_hbm, zeros_hbm, out_hbm,
            src_v, dst_v, bufs, acc, sem_ga, sem_gb, sem_sa, sem_sb):
        cid = lax.axis_index("c")
        sid = lax.axis_index("s")
        wid = cid * 16 + sid
        buf_a, buf_b = bufs[0:ss], bufs[ss:2 * ss]

        @pl.when(cid == 0)
        def _():
            pltpu.sync_copy(feat_hbm.at[pl.ds(sid * ROWS_PT, ROWS_PT)],
                            acc.at[pl.ds(sid * ROWS_PT, ROWS_PT)])

        @pl.when(cid == 1)
        def _():
            pltpu.sync_copy(zeros_hbm.at[pl.ds(sid * ROWS_PT, ROWS_PT)],
                            acc.at[pl.ds(sid * ROWS_PT, ROWS_PT)])

        pltpu.sync_copy(src_hbm.at[wid], src_v)
        pltpu.sync_copy(dst_hbm.at[wid], dst_v)
        for b in range(ss):
            pltpu.async_copy(feat_hbm.at[src_v.at[b]], buf_a[b], sem_ga)
        for b in range(ss):
            pltpu.async_copy(feat_hbm.at[src_v.at[ss + b]], buf_b[b], sem_gb)
        plsc.subcore_barrier()

        def body(g, carry):
            c = 2 * ss * g
            for b in range(ss):
                pltpu.make_async_copy(feat_hbm.at[src_v.at[0]], buf_a[b],
                                      sem_ga).wait()
                pltpu.async_copy(buf_a[b], acc.at[dst_v.at[c + b]], sem_sa,
                                 add=True)
            for b in range(ss):
                pltpu.make_async_copy(feat_hbm.at[src_v.at[0]], buf_b[b],
                                      sem_gb).wait()
                pltpu.async_copy(buf_b[b], acc.at[dst_v.at[c + ss + b]],
                                 sem_sb, add=True)
            for b in range(ss):
                pltpu.make_async_copy(buf_a[b], acc.at[dst_v.at[0]],
                                      sem_sa).wait()
            for b in range(ss):
                nxt = jnp.minimum(c + 2 * ss + b, nch - 1)
                pltpu.async_copy(feat_hbm.at[src_v.at[nxt]], buf_a[b], sem_ga)
            for b in range(ss):
                pltpu.make_async_copy(buf_b[b], acc.at[dst_v.at[0]],
                                      sem_sb).wait()
            for b in range(ss):
                nxt = jnp.minimum(c + 3 * ss + b, nch - 1)
                pltpu.async_copy(feat_hbm.at[src_v.at[nxt]], buf_b[b], sem_gb)
            return carry

        lax.fori_loop(0, nch // (2 * ss), body, 0)
        for b in range(ss):
            pltpu.make_async_copy(feat_hbm.at[src_v.at[0]], buf_a[b],
                                  sem_ga).wait()
            pltpu.make_async_copy(feat_hbm.at[src_v.at[0]], buf_b[b],
                                  sem_gb).wait()
        plsc.subcore_barrier()
        pltpu.sync_copy(acc.at[pl.ds(sid * ROWS_PT, ROWS_PT)],
                        out_hbm.at[cid, pl.ds(sid * ROWS_PT, ROWS_PT)])

    return agg


def _scale_body(d_ref, x_ref, s_ref, xs_ref):
    deg = d_ref[0][:, 0:1] + d_ref[1][:, 0:1] + 1.0
    s = lax.rsqrt(deg)
    s_ref[...] = s
    xs_ref[0:N] = x_ref[...] * s[0:N]
    xs_ref[N:NP] = jnp.zeros((NP - N, D_FEAT), jnp.float32)


def _mlp_body(a_ref, s_ref, w1_ref, b1_ref, w2_ref, ps_ref):
    s = s_ref[...]
    hin = (a_ref[0] + a_ref[1]) * s
    h = jnp.dot(hin, w1_ref[...], preferred_element_type=jnp.float32)
    h = jnp.maximum(h + b1_ref[...], 0.0)
    p = jnp.dot(h, w2_ref[...], preferred_element_type=jnp.float32)
    ps_ref[...] = p * s


def _out_body(a_ref, s_ref, b2_ref, o_ref):
    o_ref[...] = ((a_ref[0, :N] + a_ref[1, :N]) * s_ref[:N] + b2_ref[...])


def kernel(x, edge_index, W1, b1, W2, b2):
    src = edge_index[0]
    dst = edge_index[1]
    padi = (jnp.arange(EP - E, dtype=jnp.int32) % (NP - N)) + N
    srcf = jnp.concatenate([src, padi])
    dstf = jnp.concatenate([dst, padi])
    srcp1 = srcf.reshape(NTILES, EP // NTILES // 64, 64)
    dstp1 = dstf.reshape(NTILES, EP // NTILES // 64, 64)
    srcp2 = srcf.reshape(NTILES, NCHUNK, 128)
    dstp2 = dstf.reshape(NTILES, NCHUNK, 128)
    zf = jnp.zeros((NP, D_FEAT), jnp.float32)
    z16 = jnp.zeros((NP, 16), jnp.float32)
    zo = jnp.zeros((NP, D_OUT), jnp.float32)

    deg = _make_deg()(dstp2, jnp.ones((128, 16), jnp.float32), z16)

    s_col, xs = pl.pallas_call(
        _scale_body,
        out_shape=[
            jax.ShapeDtypeStruct((NP, 1), jnp.float32),
            jax.ShapeDtypeStruct((NP, D_FEAT), jnp.float32),
        ],
    )(deg, x)

    agg1 = _make_agg(D_FEAT, 64, 2)(srcp1, dstp1, xs, zf)

    ps = pl.pallas_call(
        _mlp_body,
        grid=(NP // MBLK,),
        in_specs=[
            pl.BlockSpec((2, MBLK, D_FEAT), lambda i: (0, i, 0)),
            pl.BlockSpec((MBLK, 1), lambda i: (i, 0)),
            pl.BlockSpec((D_FEAT, D_HID), lambda i: (0, 0)),
            pl.BlockSpec((1, D_HID), lambda i: (0, 0)),
            pl.BlockSpec((D_HID, D_OUT), lambda i: (0, 0)),
        ],
        out_specs=pl.BlockSpec((MBLK, D_OUT), lambda i: (i, 0)),
        out_shape=jax.ShapeDtypeStruct((NP, D_OUT), jnp.float32),
    )(agg1, s_col, W1, b1.reshape(1, D_HID), W2)

    agg2 = _make_agg(D_OUT, 128, 3)(srcp2, dstp2, ps, zo)

    out = pl.pallas_call(
        _out_body,
        out_shape=jax.ShapeDtypeStruct((N, D_OUT), jnp.float32),
    )(agg2, s_col, b2.reshape(1, D_OUT))

    return out


# speedup vs baseline: 1.0212x; 1.0054x over previous
import functools

import jax
import jax.numpy as jnp
from jax import lax
from jax.experimental import pallas as pl
from jax.experimental.pallas import tpu as pltpu
from jax.experimental.pallas import tpu_sc as plsc

N = 10000
NP = 10240
E = 160000
EP = 172032
NTILES = 32
NCHUNK = EP // NTILES // 128
ROWS_PT = NP // 16
D_FEAT = 128
D_HID = 2048
D_OUT = 64
BLK = 512
GRID = NP // BLK
MBLK = 2048

_SC_PARAMS = pltpu.CompilerParams(use_tc_tiling_on_sc=False)


@functools.cache
def _make_deg():
    @functools.partial(
        pl.kernel,
        out_type=jax.ShapeDtypeStruct((2, NP, 16), jnp.float32),
        mesh=plsc.VectorSubcoreMesh(core_axis_name="c", subcore_axis_name="s"),
        compiler_params=_SC_PARAMS,
        scratch_types=[
            pltpu.VMEM((NCHUNK, 128), jnp.int32),
            pltpu.VMEM((128, 16), jnp.float32),
            pltpu.VMEM_SHARED((NP, 16), jnp.float32),
            pltpu.SemaphoreType.DMA,
        ],
    )
    def deg_kernel(dst_hbm, ones_hbm, zeros_hbm, out_hbm, dst_v, ones_v, acc,
                   sem_s):
        cid = lax.axis_index("c")
        sid = lax.axis_index("s")
        wid = cid * 16 + sid
        pltpu.sync_copy(zeros_hbm.at[pl.ds(sid * ROWS_PT, ROWS_PT)],
                        acc.at[pl.ds(sid * ROWS_PT, ROWS_PT)])
        pltpu.sync_copy(dst_hbm.at[wid], dst_v)
        pltpu.sync_copy(ones_hbm, ones_v)
        plsc.subcore_barrier()
        for b in range(6):
            pltpu.async_copy(ones_v, acc.at[dst_v.at[b]], sem_s, add=True)

        def body(g, carry):
            c = 6 * g
            for b in range(6):
                pltpu.make_async_copy(ones_v, acc.at[dst_v.at[0]], sem_s).wait()
                pltpu.async_copy(ones_v, acc.at[dst_v.at[c + 6 + b]], sem_s,
                                 add=True)
            return carry

        lax.fori_loop(0, NCHUNK // 6 - 1, body, 0)
        for b in range(6):
            pltpu.make_async_copy(ones_v, acc.at[dst_v.at[0]], sem_s).wait()
        plsc.subcore_barrier()
        pltpu.sync_copy(acc.at[pl.ds(sid * ROWS_PT, ROWS_PT)],
                        out_hbm.at[cid, pl.ds(sid * ROWS_PT, ROWS_PT)])

    return deg_kernel


@functools.cache
def _make_agg(depth, ch, ss):
    nch = EP // NTILES // ch

    @functools.partial(
        pl.kernel,
        out_type=jax.ShapeDtypeStruct((2, NP, depth), jnp.float32),
        mesh=plsc.VectorSubcoreMesh(core_axis_name="c", subcore_axis_name="s"),
        compiler_params=_SC_PARAMS,
        scratch_types=[
            pltpu.VMEM((nch, ch), jnp.int32),
            pltpu.VMEM((nch, ch), jnp.int32),
            [pltpu.VMEM((ch, depth), jnp.float32) for _ in range(2 * ss)],
            pltpu.VMEM_SHARED((NP, depth), jnp.float32),
            pltpu.SemaphoreType.DMA,
            pltpu.SemaphoreType.DMA,
            pltpu.SemaphoreType.DMA,
            pltpu.SemaphoreType.DMA,
        ],
    )
    def agg(src_hbm, dst_hbm, feat_hbm, zeros_hbm, out_hbm,
            src_v, dst_v, bufs, acc, sem_ga, sem_gb, sem_sa, sem_sb):
        cid = lax.axis_index("c")
        sid = lax.axis_index("s")
        wid = cid * 16 + sid
        buf_a, buf_b = bufs[0:ss], bufs[ss:2 * ss]

        @pl.when(cid == 0)
        def _():
            pltpu.sync_copy(feat_hbm.at[pl.ds(sid * ROWS_PT, ROWS_PT)],
                            acc.at[pl.ds(sid * ROWS_PT, ROWS_PT)])

        @pl.when(cid == 1)
        def _():
            pltpu.sync_copy(zeros_hbm.at[pl.ds(sid * ROWS_PT, ROWS_PT)],
                            acc.at[pl.ds(sid * ROWS_PT, ROWS_PT)])

        pltpu.sync_copy(src_hbm.at[wid], src_v)
        pltpu.sync_copy(dst_hbm.at[wid], dst_v)
        for b in range(ss):
            pltpu.async_copy(feat_hbm.at[src_v.at[b]], buf_a[b], sem_ga)
        for b in range(ss):
            pltpu.async_copy(feat_hbm.at[src_v.at[ss + b]], buf_b[b], sem_gb)
        plsc.subcore_barrier()

        def body(g, carry):
            c = 2 * ss * g
            for b in range(ss):
                pltpu.make_async_copy(feat_hbm.at[src_v.at[0]], buf_a[b],
                                      sem_ga).wait()
                pltpu.async_copy(buf_a[b], acc.at[dst_v.at[c + b]], sem_sa,
                                 add=True)
            for b in range(ss):
                pltpu.make_async_copy(feat_hbm.at[src_v.at[0]], buf_b[b],
                                      sem_gb).wait()
                pltpu.async_copy(buf_b[b], acc.at[dst_v.at[c + ss + b]],
                                 sem_sb, add=True)
            for b in range(ss):
                pltpu.make_async_copy(buf_a[b], acc.at[dst_v.at[0]],
                                      sem_sa).wait()
            for b in range(ss):
                nxt = jnp.minimum(c + 2 * ss + b, nch - 1)
                pltpu.async_copy(feat_hbm.at[src_v.at[nxt]], buf_a[b], sem_ga)
            for b in range(ss):
                pltpu.make_async_copy(buf_b[b], acc.at[dst_v.at[0]],
                                      sem_sb).wait()
            for b in range(ss):
                nxt = jnp.minimum(c + 3 * ss + b, nch - 1)
                pltpu.async_copy(feat_hbm.at[src_v.at[nxt]], buf_b[b], sem_gb)
            return carry

        lax.fori_loop(0, nch // (2 * ss), body, 0)
        for b in range(ss):
            pltpu.make_async_copy(feat_hbm.at[src_v.at[0]], buf_a[b],
                                  sem_ga).wait()
            pltpu.make_async_copy(feat_hbm.at[src_v.at[0]], buf_b[b],
                                  sem_gb).wait()
        plsc.subcore_barrier()
        pltpu.sync_copy(acc.at[pl.ds(sid * ROWS_PT, ROWS_PT)],
                        out_hbm.at[cid, pl.ds(sid * ROWS_PT, ROWS_PT)])

    return agg


def _scale_body(d_ref, x_ref, s_ref, xs_ref):
    deg = d_ref[0][:, 0:1] + d_ref[1][:, 0:1] + 1.0
    s = lax.rsqrt(deg)
    s_ref[...] = s
    xs_ref[0:N] = x_ref[...] * s[0:N]
    xs_ref[N:NP] = jnp.zeros((NP - N, D_FEAT), jnp.float32)


def _mlp_body(a_ref, s_ref, w1_ref, b1_ref, w2_ref, ps_ref):
    s = s_ref[...]
    hin = (a_ref[0] + a_ref[1]) * s
    h = jnp.dot(hin, w1_ref[...], preferred_element_type=jnp.float32)
    h = jnp.maximum(h + b1_ref[...], 0.0)
    p = jnp.dot(h, w2_ref[...], preferred_element_type=jnp.float32)
    ps_ref[...] = p * s


def _out_body(a_ref, s_ref, b2_ref, o_ref):
    o_ref[...] = ((a_ref[0, :N] + a_ref[1, :N]) * s_ref[:N] + b2_ref[...])


def kernel(x, edge_index, W1, b1, W2, b2):
    src = edge_index[0]
    dst = edge_index[1]
    padi = (jnp.arange(EP - E, dtype=jnp.int32) % (NP - N)) + N
    srcf = jnp.concatenate([src, padi])
    dstf = jnp.concatenate([dst, padi])
    srcp1 = srcf.reshape(NTILES, EP // NTILES // 64, 64)
    dstp1 = dstf.reshape(NTILES, EP // NTILES // 64, 64)
    srcp2 = srcf.reshape(NTILES, NCHUNK, 128)
    dstp2 = dstf.reshape(NTILES, NCHUNK, 128)
    zf = jnp.zeros((NP, D_FEAT), jnp.float32)
    z16 = jnp.zeros((NP, 16), jnp.float32)
    zo = jnp.zeros((NP, D_OUT), jnp.float32)

    deg = _make_deg()(dstp2, jnp.ones((128, 16), jnp.float32), z16)

    s_col, xs = pl.pallas_call(
        _scale_body,
        out_shape=[
            jax.ShapeDtypeStruct((NP, 1), jnp.float32),
            jax.ShapeDtypeStruct((NP, D_FEAT), jnp.float32),
        ],
    )(deg, x)

    agg1 = _make_agg(D_FEAT, 64, 2)(srcp1, dstp1, xs, zf)

    ps = pl.pallas_call(
        _mlp_body,
        grid=(NP // MBLK,),
        in_specs=[
            pl.BlockSpec((2, MBLK, D_FEAT), lambda i: (0, i, 0)),
            pl.BlockSpec((MBLK, 1), lambda i: (i, 0)),
            pl.BlockSpec((D_FEAT, D_HID), lambda i: (0, 0)),
            pl.BlockSpec((1, D_HID), lambda i: (0, 0)),
            pl.BlockSpec((D_HID, D_OUT), lambda i: (0, 0)),
        ],
        out_specs=pl.BlockSpec((MBLK, D_OUT), lambda i: (i, 0)),
        out_shape=jax.ShapeDtypeStruct((NP, D_OUT), jnp.float32),
    )(agg1, s_col, W1, b1.reshape(1, D_HID), W2)

    agg2 = _make_agg(D_OUT, 128, 3)(srcp2, dstp2, ps, zo)

    out = pl.pallas_call(
        _out_body,
        out_shape=jax.ShapeDtypeStruct((N, D_OUT), jnp.float32),
    )(agg2, s_col, b2.reshape(1, D_OUT))

    return out


# speedup vs baseline: 1.0472x; 1.0254x over previous
import functools

import jax
import jax.numpy as jnp
from jax import lax
from jax.experimental import pallas as pl
from jax.experimental.pallas import tpu as pltpu
from jax.experimental.pallas import tpu_sc as plsc

N = 10000
NP = 10240
E = 160000
EP = 172032
NTILES = 32
NCHUNK = EP // NTILES // 128
ROWS_PT = NP // 16
D_FEAT = 128
D_HID = 2048
D_OUT = 64
BLK = 512
GRID = NP // BLK
MBLK = 2048

_SC_PARAMS = pltpu.CompilerParams(use_tc_tiling_on_sc=False)


@functools.cache
def _make_deg():
    @functools.partial(
        pl.kernel,
        out_type=jax.ShapeDtypeStruct((2, NP, 16), jnp.float32),
        mesh=plsc.VectorSubcoreMesh(core_axis_name="c", subcore_axis_name="s"),
        compiler_params=_SC_PARAMS,
        scratch_types=[
            pltpu.VMEM((NCHUNK, 128), jnp.int32),
            pltpu.VMEM((128, 16), jnp.float32),
            pltpu.VMEM_SHARED((NP, 16), jnp.float32),
            pltpu.SemaphoreType.DMA,
        ],
    )
    def deg_kernel(edges_hbm, ones_hbm, zeros_hbm, out_hbm, dst_v, ones_v,
                   acc, sem_s):
        cid = lax.axis_index("c")
        sid = lax.axis_index("s")
        wid = cid * 16 + sid
        pltpu.sync_copy(zeros_hbm.at[pl.ds(sid * ROWS_PT, ROWS_PT)],
                        acc.at[pl.ds(sid * ROWS_PT, ROWS_PT)])
        pltpu.sync_copy(edges_hbm.at[1, wid], dst_v)
        pltpu.sync_copy(ones_hbm, ones_v)
        plsc.subcore_barrier()
        for b in range(6):
            pltpu.async_copy(ones_v, acc.at[dst_v.at[b]], sem_s, add=True)

        def body(g, carry):
            c = 6 * g
            for b in range(6):
                pltpu.make_async_copy(ones_v, acc.at[dst_v.at[0]], sem_s).wait()
                pltpu.async_copy(ones_v, acc.at[dst_v.at[c + 6 + b]], sem_s,
                                 add=True)
            return carry

        lax.fori_loop(0, NCHUNK // 6 - 1, body, 0)
        for b in range(6):
            pltpu.make_async_copy(ones_v, acc.at[dst_v.at[0]], sem_s).wait()
        plsc.subcore_barrier()
        pltpu.sync_copy(acc.at[pl.ds(sid * ROWS_PT, ROWS_PT)],
                        out_hbm.at[cid, pl.ds(sid * ROWS_PT, ROWS_PT)])

    return deg_kernel


@functools.cache
def _make_agg(depth, ch, ss):
    nch = EP // NTILES // ch

    @functools.partial(
        pl.kernel,
        out_type=jax.ShapeDtypeStruct((2, NP, depth), jnp.float32),
        mesh=plsc.VectorSubcoreMesh(core_axis_name="c", subcore_axis_name="s"),
        compiler_params=_SC_PARAMS,
        scratch_types=[
            pltpu.VMEM((nch, ch), jnp.int32),
            pltpu.VMEM((nch, ch), jnp.int32),
            [pltpu.VMEM((ch, depth), jnp.float32) for _ in range(2 * ss)],
            pltpu.VMEM_SHARED((NP, depth), jnp.float32),
            pltpu.SemaphoreType.DMA,
            pltpu.SemaphoreType.DMA,
            pltpu.SemaphoreType.DMA,
            pltpu.SemaphoreType.DMA,
        ],
    )
    def agg(edges_hbm, feat_hbm, zeros_hbm, out_hbm,
            src_v, dst_v, bufs, acc, sem_ga, sem_gb, sem_sa, sem_sb):
        cid = lax.axis_index("c")
        sid = lax.axis_index("s")
        wid = cid * 16 + sid
        buf_a, buf_b = bufs[0:ss], bufs[ss:2 * ss]

        @pl.when(cid == 0)
        def _():
            pltpu.sync_copy(feat_hbm.at[pl.ds(sid * ROWS_PT, ROWS_PT)],
                            acc.at[pl.ds(sid * ROWS_PT, ROWS_PT)])

        @pl.when(cid == 1)
        def _():
            pltpu.sync_copy(zeros_hbm.at[pl.ds(sid * ROWS_PT, ROWS_PT)],
                            acc.at[pl.ds(sid * ROWS_PT, ROWS_PT)])

        pltpu.sync_copy(edges_hbm.at[0, wid], src_v)
        pltpu.sync_copy(edges_hbm.at[1, wid], dst_v)
        for b in range(ss):
            pltpu.async_copy(feat_hbm.at[src_v.at[b]], buf_a[b], sem_ga)
        for b in range(ss):
            pltpu.async_copy(feat_hbm.at[src_v.at[ss + b]], buf_b[b], sem_gb)
        plsc.subcore_barrier()

        def body(g, carry):
            c = 2 * ss * g
            for b in range(ss):
                pltpu.make_async_copy(feat_hbm.at[src_v.at[0]], buf_a[b],
                                      sem_ga).wait()
                pltpu.async_copy(buf_a[b], acc.at[dst_v.at[c + b]], sem_sa,
                                 add=True)
            for b in range(ss):
                pltpu.make_async_copy(feat_hbm.at[src_v.at[0]], buf_b[b],
                                      sem_gb).wait()
                pltpu.async_copy(buf_b[b], acc.at[dst_v.at[c + ss + b]],
                                 sem_sb, add=True)
            for b in range(ss):
                pltpu.make_async_copy(buf_a[b], acc.at[dst_v.at[0]],
                                      sem_sa).wait()
            for b in range(ss):
                nxt = jnp.minimum(c + 2 * ss + b, nch - 1)
                pltpu.async_copy(feat_hbm.at[src_v.at[nxt]], buf_a[b], sem_ga)
            for b in range(ss):
                pltpu.make_async_copy(buf_b[b], acc.at[dst_v.at[0]],
                                      sem_sb).wait()
            for b in range(ss):
                nxt = jnp.minimum(c + 3 * ss + b, nch - 1)
                pltpu.async_copy(feat_hbm.at[src_v.at[nxt]], buf_b[b], sem_gb)
            return carry

        lax.fori_loop(0, nch // (2 * ss), body, 0)
        for b in range(ss):
            pltpu.make_async_copy(feat_hbm.at[src_v.at[0]], buf_a[b],
                                  sem_ga).wait()
            pltpu.make_async_copy(feat_hbm.at[src_v.at[0]], buf_b[b],
                                  sem_gb).wait()
        plsc.subcore_barrier()
        pltpu.sync_copy(acc.at[pl.ds(sid * ROWS_PT, ROWS_PT)],
                        out_hbm.at[cid, pl.ds(sid * ROWS_PT, ROWS_PT)])

    return agg


def _scale_body(d_ref, x_ref, s_ref, xs_ref):
    deg = d_ref[0][:, 0:1] + d_ref[1][:, 0:1] + 1.0
    s = lax.rsqrt(deg)
    s_ref[...] = s
    xs_ref[0:N] = x_ref[...] * s[0:N]
    xs_ref[N:NP] = jnp.zeros((NP - N, D_FEAT), jnp.float32)


def _mlp_body(a_ref, s_ref, w1_ref, b1_ref, w2_ref, ps_ref):
    s = s_ref[...]
    hin = (a_ref[0] + a_ref[1]) * s
    h = jnp.dot(hin, w1_ref[...], preferred_element_type=jnp.float32)
    h = jnp.maximum(h + b1_ref[...], 0.0)
    p = jnp.dot(h, w2_ref[...], preferred_element_type=jnp.float32)
    ps_ref[...] = p * s


def _out_body(a_ref, s_ref, b2_ref, o_ref):
    o_ref[...] = ((a_ref[0, :N] + a_ref[1, :N]) * s_ref[:N] + b2_ref[...])


def kernel(x, edge_index, W1, b1, W2, b2):
    padi = (jnp.arange(EP - E, dtype=jnp.int32) % (NP - N)) + N
    edges = jnp.concatenate(
        [edge_index, jnp.broadcast_to(padi[None, :], (2, EP - E))], axis=1)
    edges1 = edges.reshape(2, NTILES, EP // NTILES // 64, 64)
    edges2 = edges.reshape(2, NTILES, NCHUNK, 128)
    zf = jnp.zeros((NP, D_FEAT), jnp.float32)
    z16 = jnp.zeros((NP, 16), jnp.float32)
    zo = jnp.zeros((NP, D_OUT), jnp.float32)

    deg = _make_deg()(edges2, jnp.ones((128, 16), jnp.float32), z16)

    s_col, xs = pl.pallas_call(
        _scale_body,
        out_shape=[
            jax.ShapeDtypeStruct((NP, 1), jnp.float32),
            jax.ShapeDtypeStruct((NP, D_FEAT), jnp.float32),
        ],
    )(deg, x)

    agg1 = _make_agg(D_FEAT, 64, 2)(edges1, xs, zf)

    ps = pl.pallas_call(
        _mlp_body,
        grid=(NP // MBLK,),
        in_specs=[
            pl.BlockSpec((2, MBLK, D_FEAT), lambda i: (0, i, 0)),
            pl.BlockSpec((MBLK, 1), lambda i: (i, 0)),
            pl.BlockSpec((D_FEAT, D_HID), lambda i: (0, 0)),
            pl.BlockSpec((1, D_HID), lambda i: (0, 0)),
            pl.BlockSpec((D_HID, D_OUT), lambda i: (0, 0)),
        ],
        out_specs=pl.BlockSpec((MBLK, D_OUT), lambda i: (i, 0)),
        out_shape=jax.ShapeDtypeStruct((NP, D_OUT), jnp.float32),
    )(agg1, s_col, W1, b1.reshape(1, D_HID), W2)

    agg2 = _make_agg(D_OUT, 128, 3)(edges2, ps, zo)

    out = pl.pallas_call(
        _out_body,
        out_shape=jax.ShapeDtypeStruct((N, D_OUT), jnp.float32),
    )(agg2, s_col, b2.reshape(1, D_OUT))

    return out
